# Initial kernel scaffold; baseline (speedup 1.0000x reference)
import jax
import jax.numpy as jnp
from jax.experimental import pallas as pl


def kernel(x, y, mask, th_per_class, W):
    raise NotImplementedError("write your pallas kernel here")


# speedup vs baseline: 1.2968x; 1.2968x over previous
import functools

import jax
import jax.numpy as jnp
from jax.experimental import pallas as pl
from jax.experimental.pallas import tpu as pltpu

_LAMBDA_DECAY = 0.99
_MIN_TH = 0.8
_MAX_TH = 0.95
_P = 2


def _colsum_body(y_ref, cs_ref):
    j = pl.program_id(1)
    y = y_ref[...]
    m = jnp.max(y, axis=1, keepdims=True)
    e = jnp.exp(y - m)
    z = jnp.sum(e, axis=1, keepdims=True)
    probs = e / z

    @pl.when(j == 0)
    def _():
        cs_ref[...] = jnp.zeros_like(cs_ref)

    cs_ref[...] += jnp.sum(probs, axis=0, keepdims=True)[None]


def _main_body(x_ref, y_ref, w_ref, mask_ref, cs_ref, thpc_ref,
               loss_ref, freq_ref, *, inv_b):
    j = pl.program_id(1)

    mean = jnp.sum(cs_ref[...], axis=0) * inv_b
    th = thpc_ref[...] * _LAMBDA_DECAY + (1.0 - _LAMBDA_DECAY) * mean
    tmin = jnp.min(th, axis=1, keepdims=True)
    tmax = jnp.max(th, axis=1, keepdims=True)
    th = (th - tmin) / (tmax - tmin) * (_MAX_TH - _MIN_TH) + _MIN_TH

    y = y_ref[...]
    m = jnp.max(y, axis=1, keepdims=True)
    e = jnp.exp(y - m)
    z = jnp.sum(e, axis=1, keepdims=True)
    probs = e / z

    gt = (probs > th).astype(jnp.float32)
    gtm = jnp.max(gt, axis=1, keepdims=True)

    pm = jnp.max(probs, axis=1, keepdims=True)
    onehot = (probs == pm).astype(jnp.float32)
    p_t = jnp.where(gtm > 0.0, 10.0 * onehot, probs)

    out = jnp.dot(x_ref[...], w_ref[...], preferred_element_type=jnp.float32)
    om = jnp.max(out, axis=1, keepdims=True)
    oe = jnp.exp(out - om)
    oz = jnp.sum(oe, axis=1, keepdims=True)
    logsm = (out - om) - jnp.log(oz)

    rowloss = -jnp.sum(p_t * logsm, axis=1, keepdims=True)
    lsum = jnp.sum(rowloss * mask_ref[...], axis=0, keepdims=True)

    fpart = jnp.sum(gt, axis=0, keepdims=True)
    nofire = jnp.sum(1.0 - gtm, axis=0, keepdims=True)
    lane = jax.lax.broadcasted_iota(jnp.int32, fpart.shape, 1)
    fpart = fpart + jnp.where(lane == 0, nofire, 0.0)

    @pl.when(j == 0)
    def _():
        loss_ref[...] = jnp.zeros_like(loss_ref)
        freq_ref[...] = jnp.zeros_like(freq_ref)

    loss_ref[...] += lsum[None]
    freq_ref[...] += fpart[None]


def kernel(x, y, mask, th_per_class, W):
    B, D = x.shape
    C = y.shape[1]
    mask2 = mask.reshape(B, 1)
    thpc2 = th_per_class.reshape(1, C)

    bb1 = 1024
    nb1 = B // (_P * bb1)
    cs = pl.pallas_call(
        _colsum_body,
        grid=(_P, nb1),
        in_specs=[pl.BlockSpec((bb1, C), lambda i, j: (i * nb1 + j, 0))],
        out_specs=pl.BlockSpec((1, 1, C), lambda i, j: (i, 0, 0)),
        out_shape=jax.ShapeDtypeStruct((_P, 1, C), jnp.float32),
        compiler_params=pltpu.CompilerParams(
            dimension_semantics=("parallel", "arbitrary"),
            vmem_limit_bytes=56 * 1024 * 1024,
        ),
    )(y)

    bb2 = 512
    nb2 = B // (_P * bb2)
    loss_parts, freq_parts = pl.pallas_call(
        functools.partial(_main_body, inv_b=1.0 / B),
        grid=(_P, nb2),
        in_specs=[
            pl.BlockSpec((bb2, D), lambda i, j: (i * nb2 + j, 0)),
            pl.BlockSpec((bb2, C), lambda i, j: (i * nb2 + j, 0)),
            pl.BlockSpec((D, C), lambda i, j: (0, 0)),
            pl.BlockSpec((bb2, 1), lambda i, j: (i * nb2 + j, 0)),
            pl.BlockSpec((_P, 1, C), lambda i, j: (0, 0, 0)),
            pl.BlockSpec((1, C), lambda i, j: (0, 0)),
        ],
        out_specs=[
            pl.BlockSpec((1, 1, 1), lambda i, j: (i, 0, 0)),
            pl.BlockSpec((1, 1, C), lambda i, j: (i, 0, 0)),
        ],
        out_shape=[
            jax.ShapeDtypeStruct((_P, 1, 1), jnp.float32),
            jax.ShapeDtypeStruct((_P, 1, C), jnp.float32),
        ],
        compiler_params=pltpu.CompilerParams(
            dimension_semantics=("parallel", "arbitrary"),
            vmem_limit_bytes=56 * 1024 * 1024,
        ),
    )(x, y, W, mask2, cs, thpc2)

    loss = jnp.sum(loss_parts) / B
    class_freq = jnp.sum(freq_parts, axis=(0, 1))
    return (loss, class_freq)
